# Initial kernel scaffold; baseline (speedup 1.0000x reference)
import jax
import jax.numpy as jnp
from jax.experimental import pallas as pl


def kernel(inputs, word_emb, seg_emb, pos_emb, gamma, beta):
    raise NotImplementedError("write your pallas kernel here")


# speedup vs baseline: 3.1164x; 3.1164x over previous
import functools

import jax
import jax.numpy as jnp
from jax import lax
from jax.experimental import pallas as pl
from jax.experimental.pallas import tpu as pltpu
from jax.experimental.pallas import tpu_sc as plsc

B, L, E = 128, 512, 768
NC, NS = 2, 16
NW = NC * NS
TPW = (B * L) // NW
R = 64
NCH = TPW // R
TYPE_ROWS = 2


def _sc_gather(ids_flat, word_emb):
    mesh = plsc.VectorSubcoreMesh(core_axis_name="c", subcore_axis_name="s")

    @functools.partial(
        pl.kernel,
        mesh=mesh,
        out_type=jax.ShapeDtypeStruct((B * L, E), jnp.float32),
        scratch_types=[
            pltpu.VMEM((TPW,), jnp.int32),
            pltpu.VMEM((2, R, E), jnp.float32),
            pltpu.SemaphoreType.DMA,
            pltpu.SemaphoreType.DMA,
            pltpu.SemaphoreType.DMA,
            pltpu.SemaphoreType.DMA,
        ],
    )
    def k(ids_hbm, word_hbm, out_hbm, idx_v, rows_v, g0, g1, s0, s1):
        wid = lax.axis_index("s") * NC + lax.axis_index("c")
        base = pl.multiple_of(wid * TPW, TPW)
        pltpu.sync_copy(ids_hbm.at[pl.ds(base, TPW)], idx_v)
        gs = (g0, g1)
        ss = (s0, s1)

        def gather(c, slot):
            off = pl.multiple_of(c * R, R)
            return pltpu.make_async_copy(
                word_hbm.at[idx_v.at[pl.ds(off, R)]], rows_v.at[slot], gs[slot]
            )

        def store(c, slot):
            off = pl.multiple_of(base + c * R, R)
            return pltpu.make_async_copy(
                rows_v.at[slot], out_hbm.at[pl.ds(off, R)], ss[slot]
            )

        gather(0, 0).start()
        gather(1, 1).start()

        def step(i, carry):
            c0 = 2 * i
            gather(c0, 0).wait()
            store(c0, 0).start()
            gather(c0 + 1, 1).wait()
            store(c0 + 1, 1).start()
            store(c0, 0).wait()

            @pl.when(c0 + 2 < NCH)
            def _():
                gather(c0 + 2, 0).start()

            store(c0 + 1, 1).wait()

            @pl.when(c0 + 3 < NCH)
            def _():
                gather(c0 + 3, 1).start()

            return carry

        lax.fori_loop(0, NCH // 2, step, 0)

    return k(ids_flat, word_emb)


def _tc_layernorm(gathered, pos_emb, seg_emb, gamma, beta):
    def body(x_ref, pos_ref, seg_ref, g_ref, b_ref, o_ref):
        x = x_ref[...] + pos_ref[...] + seg_ref[0, :][None, :]
        mean = jnp.mean(x, axis=-1, keepdims=True)
        xc = x - mean
        var = jnp.mean(xc * xc, axis=-1, keepdims=True)
        o_ref[...] = xc * lax.rsqrt(var + 1e-12) * g_ref[0, :][None, :] + b_ref[0, :][None, :]

    return pl.pallas_call(
        body,
        grid=(B,),
        in_specs=[
            pl.BlockSpec((L, E), lambda i: (i, 0)),
            pl.BlockSpec((L, E), lambda i: (0, 0)),
            pl.BlockSpec((TYPE_ROWS, E), lambda i: (0, 0)),
            pl.BlockSpec((1, E), lambda i: (0, 0)),
            pl.BlockSpec((1, E), lambda i: (0, 0)),
        ],
        out_specs=pl.BlockSpec((L, E), lambda i: (i, 0)),
        out_shape=jax.ShapeDtypeStruct((B * L, E), jnp.float32),
    )(gathered, pos_emb, seg_emb, gamma.reshape(1, E), beta.reshape(1, E))


def kernel(inputs, word_emb, seg_emb, pos_emb, gamma, beta):
    ids_flat = inputs.reshape(-1).astype(jnp.int32)
    gathered = _sc_gather(ids_flat, word_emb)
    out = _tc_layernorm(gathered, pos_emb, seg_emb, gamma, beta)
    return out.reshape(B, L, E)


# speedup vs baseline: 3.4911x; 1.1202x over previous
import functools

import jax
import jax.numpy as jnp
from jax import lax
from jax.experimental import pallas as pl
from jax.experimental.pallas import tpu as pltpu
from jax.experimental.pallas import tpu_sc as plsc

B, L, E = 128, 512, 768
NC, NS = 2, 16
NW = NC * NS
TPW = (B * L) // NW
R = 64
NCH = TPW // R
TYPE_ROWS = 2
LN_BATCHES = 2


def _sc_gather(ids_flat, word_emb):
    mesh = plsc.VectorSubcoreMesh(core_axis_name="c", subcore_axis_name="s")

    @functools.partial(
        pl.kernel,
        mesh=mesh,
        out_type=jax.ShapeDtypeStruct((B * L, E), jnp.float32),
        scratch_types=[
            pltpu.VMEM((TPW,), jnp.int32),
            pltpu.VMEM((2, R, E), jnp.float32),
            pltpu.SemaphoreType.DMA,
            pltpu.SemaphoreType.DMA,
            pltpu.SemaphoreType.DMA,
            pltpu.SemaphoreType.DMA,
        ],
    )
    def k(ids_hbm, word_hbm, out_hbm, idx_v, rows_v, g0, g1, s0, s1):
        wid = lax.axis_index("s") * NC + lax.axis_index("c")
        base = pl.multiple_of(wid * TPW, TPW)
        pltpu.sync_copy(ids_hbm.at[pl.ds(base, TPW)], idx_v)
        gs = (g0, g1)
        ss = (s0, s1)

        def gather(c, slot):
            off = pl.multiple_of(c * R, R)
            return pltpu.make_async_copy(
                word_hbm.at[idx_v.at[pl.ds(off, R)]], rows_v.at[slot], gs[slot]
            )

        def store(c, slot):
            off = pl.multiple_of(base + c * R, R)
            return pltpu.make_async_copy(
                rows_v.at[slot], out_hbm.at[pl.ds(off, R)], ss[slot]
            )

        gather(0, 0).start()
        gather(1, 1).start()

        def step(i, carry):
            c0 = 2 * i
            gather(c0, 0).wait()
            store(c0, 0).start()
            gather(c0 + 1, 1).wait()
            store(c0 + 1, 1).start()
            store(c0, 0).wait()

            @pl.when(c0 + 2 < NCH)
            def _():
                gather(c0 + 2, 0).start()

            store(c0 + 1, 1).wait()

            @pl.when(c0 + 3 < NCH)
            def _():
                gather(c0 + 3, 1).start()

            return carry

        lax.fori_loop(0, NCH // 2, step, 0)

    return k(ids_flat, word_emb)


def _tc_layernorm(gathered, pos_emb, seg_emb, gamma, beta):
    def body(x_ref, pos_ref, seg_ref, g_ref, b_ref, o_ref):
        x = x_ref[...].reshape(LN_BATCHES, L, E) + pos_ref[...][None] + seg_ref[0, :][None, None, :]
        mean = jnp.mean(x, axis=-1, keepdims=True)
        xc = x - mean
        var = jnp.mean(xc * xc, axis=-1, keepdims=True)
        y = xc * lax.rsqrt(var + 1e-12) * g_ref[0, :][None, None, :] + b_ref[0, :][None, None, :]
        o_ref[...] = y.reshape(LN_BATCHES * L, E)

    return pl.pallas_call(
        body,
        grid=(B // LN_BATCHES,),
        in_specs=[
            pl.BlockSpec((LN_BATCHES * L, E), lambda i: (i, 0)),
            pl.BlockSpec((L, E), lambda i: (0, 0)),
            pl.BlockSpec((TYPE_ROWS, E), lambda i: (0, 0)),
            pl.BlockSpec((1, E), lambda i: (0, 0)),
            pl.BlockSpec((1, E), lambda i: (0, 0)),
        ],
        out_specs=pl.BlockSpec((LN_BATCHES * L, E), lambda i: (i, 0)),
        out_shape=jax.ShapeDtypeStruct((B * L, E), jnp.float32),
    )(gathered, pos_emb, seg_emb, gamma.reshape(1, E), beta.reshape(1, E))


def kernel(inputs, word_emb, seg_emb, pos_emb, gamma, beta):
    ids_flat = inputs.reshape(-1).astype(jnp.int32)
    gathered = _sc_gather(ids_flat, word_emb)
    out = _tc_layernorm(gathered, pos_emb, seg_emb, gamma, beta)
    return out.reshape(B, L, E)


# speedup vs baseline: 3.6440x; 1.0438x over previous
import functools

import jax
import jax.numpy as jnp
from jax import lax
from jax.experimental import pallas as pl
from jax.experimental.pallas import tpu as pltpu
from jax.experimental.pallas import tpu_sc as plsc

B, L, E = 128, 512, 768
NC, NS = 2, 16
NW = NC * NS
TPW = (B * L) // NW
R = 64
NCH = TPW // R
TYPE_ROWS = 2
LN_BATCHES = 4


def _sc_gather(ids_flat, word_emb):
    mesh = plsc.VectorSubcoreMesh(core_axis_name="c", subcore_axis_name="s")

    @functools.partial(
        pl.kernel,
        mesh=mesh,
        out_type=jax.ShapeDtypeStruct((B * L, E), jnp.float32),
        scratch_types=[
            pltpu.VMEM((TPW,), jnp.int32),
            pltpu.VMEM((2, R, E), jnp.float32),
            pltpu.SemaphoreType.DMA,
            pltpu.SemaphoreType.DMA,
            pltpu.SemaphoreType.DMA,
            pltpu.SemaphoreType.DMA,
        ],
    )
    def k(ids_hbm, word_hbm, out_hbm, idx_v, rows_v, g0, g1, s0, s1):
        wid = lax.axis_index("s") * NC + lax.axis_index("c")
        base = pl.multiple_of(wid * TPW, TPW)
        pltpu.sync_copy(ids_hbm.at[pl.ds(base, TPW)], idx_v)
        gs = (g0, g1)
        ss = (s0, s1)

        def gather(c, slot):
            off = pl.multiple_of(c * R, R)
            return pltpu.make_async_copy(
                word_hbm.at[idx_v.at[pl.ds(off, R)]], rows_v.at[slot], gs[slot]
            )

        def store(c, slot):
            off = pl.multiple_of(base + c * R, R)
            return pltpu.make_async_copy(
                rows_v.at[slot], out_hbm.at[pl.ds(off, R)], ss[slot]
            )

        gather(0, 0).start()
        gather(1, 1).start()

        def step(i, carry):
            c0 = 2 * i
            gather(c0, 0).wait()
            store(c0, 0).start()
            gather(c0 + 1, 1).wait()
            store(c0 + 1, 1).start()
            store(c0, 0).wait()

            @pl.when(c0 + 2 < NCH)
            def _():
                gather(c0 + 2, 0).start()

            store(c0 + 1, 1).wait()

            @pl.when(c0 + 3 < NCH)
            def _():
                gather(c0 + 3, 1).start()

            return carry

        lax.fori_loop(0, NCH // 2, step, 0)

    return k(ids_flat, word_emb)


def _tc_layernorm(gathered, pos_emb, seg_emb, gamma, beta):
    def body(x_ref, pos_ref, seg_ref, g_ref, b_ref, o_ref):
        x = x_ref[...].reshape(LN_BATCHES, L, E) + pos_ref[...][None] + seg_ref[0, :][None, None, :]
        mean = jnp.mean(x, axis=-1, keepdims=True)
        xc = x - mean
        var = jnp.mean(xc * xc, axis=-1, keepdims=True)
        y = xc * lax.rsqrt(var + 1e-12) * g_ref[0, :][None, None, :] + b_ref[0, :][None, None, :]
        o_ref[...] = y.reshape(LN_BATCHES * L, E)

    return pl.pallas_call(
        body,
        grid=(B // LN_BATCHES,),
        in_specs=[
            pl.BlockSpec((LN_BATCHES * L, E), lambda i: (i, 0)),
            pl.BlockSpec((L, E), lambda i: (0, 0)),
            pl.BlockSpec((TYPE_ROWS, E), lambda i: (0, 0)),
            pl.BlockSpec((1, E), lambda i: (0, 0)),
            pl.BlockSpec((1, E), lambda i: (0, 0)),
        ],
        out_specs=pl.BlockSpec((LN_BATCHES * L, E), lambda i: (i, 0)),
        out_shape=jax.ShapeDtypeStruct((B * L, E), jnp.float32),
    )(gathered, pos_emb, seg_emb, gamma.reshape(1, E), beta.reshape(1, E))


def kernel(inputs, word_emb, seg_emb, pos_emb, gamma, beta):
    ids_flat = inputs.reshape(-1).astype(jnp.int32)
    gathered = _sc_gather(ids_flat, word_emb)
    out = _tc_layernorm(gathered, pos_emb, seg_emb, gamma, beta)
    return out.reshape(B, L, E)
